# baseline (speedup 1.0000x reference)
import functools

import jax
import jax.numpy as jnp
from jax import lax
from jax.experimental import pallas as pl
from jax.experimental.pallas import tpu as pltpu
from jax.experimental.pallas import tpu_sc as plsc

EMB = 128
HID = 128

_NC, _NS = 2, 16
_NW = _NC * _NS
_NBUF = 4


_PROJ_BLOCK = 10000


def _proj_body(t_ref, w_ref, o_ref):
    i = pl.program_id(0)
    blk = t_ref[...]
    rows = lax.broadcasted_iota(jnp.int32, blk.shape, 0) + i * _PROJ_BLOCK
    blk = jnp.where(rows == 1, 0.0, blk)
    o_ref[...] = lax.dot_general(
        blk, w_ref[...], (((1,), (1,)), ((), ())),
        preferred_element_type=jnp.float32)


def _project_table(table, W):
    vocab = table.shape[0]
    grid = vocab // _PROJ_BLOCK
    return pl.pallas_call(
        _proj_body,
        grid=(grid,),
        in_specs=[
            pl.BlockSpec((_PROJ_BLOCK, EMB), lambda i: (i, 0)),
            pl.BlockSpec((HID, EMB), lambda i: (0, 0)),
        ],
        out_specs=pl.BlockSpec((_PROJ_BLOCK, HID), lambda i: (i, 0)),
        out_shape=jax.ShapeDtypeStruct((vocab, HID), jnp.float32),
    )(table, W)


@functools.partial(jax.jit, static_argnums=(2, 3))
def _gather_rows(p, idx3, batch, seq):
    n_ch = batch // _NW
    n_groups = n_ch // _NBUF
    mesh = plsc.VectorSubcoreMesh(core_axis_name="c", subcore_axis_name="s")

    @functools.partial(
        pl.kernel,
        mesh=mesh,
        out_type=jax.ShapeDtypeStruct((batch, seq, HID), jnp.float32),
        scratch_types=[
            pltpu.VMEM((n_ch, seq), jnp.int32),
        ] + [pltpu.VMEM((seq, HID), jnp.float32) for _ in range(_NBUF)]
          + [pltpu.SemaphoreType.DMA for _ in range(2 * _NBUF)],
    )
    def gather_k(p_hbm, idx_hbm, out_hbm, idx_v,
                 b0, b1, b2, b3, g0, g1, g2, g3, o0, o1, o2, o3):
        bufs = (b0, b1, b2, b3)
        gsem = (g0, g1, g2, g3)
        osem = (o0, o1, o2, o3)
        wid = lax.axis_index("s") * _NC + lax.axis_index("c")
        base = wid * n_ch
        pltpu.sync_copy(idx_hbm.at[wid], idx_v)
        for b in range(_NBUF):
            pltpu.async_copy(p_hbm.at[idx_v.at[b]], bufs[b], gsem[b])

        def group(g, carry):
            j0 = g * _NBUF
            for b in range(_NBUF):
                j = j0 + b
                dst = out_hbm.at[base + j]
                pltpu.make_async_copy(
                    p_hbm.at[idx_v.at[j]], bufs[b], gsem[b]).wait()
                pltpu.async_copy(bufs[b], dst, osem[b])

                @pl.when(g < n_groups - 1)
                def _():
                    pltpu.make_async_copy(bufs[b], dst, osem[b]).wait()
                    pltpu.async_copy(
                        p_hbm.at[idx_v.at[j + _NBUF]], bufs[b], gsem[b])
            return carry

        lax.fori_loop(0, n_groups, group, 0)
        last = (n_groups - 1) * _NBUF
        for b in range(_NBUF):
            j = last + b
            pltpu.make_async_copy(
                bufs[b], out_hbm.at[base + j], osem[b]).wait()

    return gather_k(p, idx3)


_S = 2


def kernel(prem, hypo, embed_table, W):
    B, L = prem.shape
    nslice = B // _S
    n_ch = nslice // _NW

    P = _project_table(embed_table, W)
    outs = []
    for idx in (prem, hypo):
        parts = [
            _gather_rows(
                P,
                lax.slice_in_dim(idx, s * nslice, (s + 1) * nslice)
                .reshape(_NW, n_ch, L),
                nslice, L)
            for s in range(_S)
        ]
        outs.append(jnp.concatenate(parts, axis=0))
    return (outs[0], outs[1])


# speedup vs baseline: 1.5434x; 1.5434x over previous
import functools

import jax
import jax.numpy as jnp
from jax import lax
from jax.experimental import pallas as pl
from jax.experimental.pallas import tpu as pltpu
from jax.experimental.pallas import tpu_sc as plsc

EMB = 128
HID = 128

_NC, _NS = 2, 16
_NW = _NC * _NS
_NBUF = 8


_PROJ_BLOCK = 20000


def _proj_body(t_ref, w_ref, o_ref):
    i = pl.program_id(0)
    blk = t_ref[...]
    rows = lax.broadcasted_iota(jnp.int32, blk.shape, 0) + i * _PROJ_BLOCK
    blk = jnp.where(rows == 1, 0.0, blk)
    o_ref[...] = lax.dot_general(
        blk, w_ref[...], (((1,), (1,)), ((), ())),
        preferred_element_type=jnp.float32)


def _project_table(table, W):
    vocab = table.shape[0]
    grid = vocab // _PROJ_BLOCK
    return pl.pallas_call(
        _proj_body,
        grid=(grid,),
        in_specs=[
            pl.BlockSpec((_PROJ_BLOCK, EMB), lambda i: (i, 0)),
            pl.BlockSpec((HID, EMB), lambda i: (0, 0)),
        ],
        out_specs=pl.BlockSpec((_PROJ_BLOCK, HID), lambda i: (i, 0)),
        out_shape=jax.ShapeDtypeStruct((vocab, HID), jnp.float32),
    )(table, W)


@functools.partial(jax.jit, static_argnums=(2, 3))
def _gather_rows(p, idx3, batch, seq):
    n_ch = batch // _NW
    n_groups = n_ch // _NBUF
    mesh = plsc.VectorSubcoreMesh(core_axis_name="c", subcore_axis_name="s")

    @functools.partial(
        pl.kernel,
        mesh=mesh,
        out_type=jax.ShapeDtypeStruct((batch, seq, HID), jnp.float32),
        scratch_types=[
            pltpu.VMEM((n_ch, seq), jnp.int32),
        ] + [pltpu.VMEM((seq, HID), jnp.float32) for _ in range(_NBUF)]
          + [pltpu.SemaphoreType.DMA for _ in range(2 * _NBUF)],
    )
    def gather_k(p_hbm, idx_hbm, out_hbm, idx_v, *scratch):
        bufs = scratch[:_NBUF]
        gsem = scratch[_NBUF:2 * _NBUF]
        osem = scratch[2 * _NBUF:]
        wid = lax.axis_index("s") * _NC + lax.axis_index("c")
        base = wid * n_ch
        pltpu.sync_copy(idx_hbm.at[wid], idx_v)
        for b in range(_NBUF):
            pltpu.async_copy(p_hbm.at[idx_v.at[b]], bufs[b], gsem[b])

        def group(g, carry):
            j0 = g * _NBUF
            for b in range(_NBUF):
                j = j0 + b
                dst = out_hbm.at[base + j]
                pltpu.make_async_copy(
                    p_hbm.at[idx_v.at[j]], bufs[b], gsem[b]).wait()
                pltpu.async_copy(bufs[b], dst, osem[b])

                @pl.when(g < n_groups - 1)
                def _():
                    pltpu.make_async_copy(bufs[b], dst, osem[b]).wait()
                    pltpu.async_copy(
                        p_hbm.at[idx_v.at[j + _NBUF]], bufs[b], gsem[b])
            return carry

        lax.fori_loop(0, n_groups, group, 0)
        last = (n_groups - 1) * _NBUF
        for b in range(_NBUF):
            j = last + b
            pltpu.make_async_copy(
                bufs[b], out_hbm.at[base + j], osem[b]).wait()

    return gather_k(p, idx3)


def kernel(prem, hypo, embed_table, W):
    B, L = prem.shape
    n_ch = B // _NW

    P = _project_table(embed_table, W)
    outs = [
        _gather_rows(P, idx.reshape(_NW, n_ch, L), B, L)
        for idx in (prem, hypo)
    ]
    return (outs[0], outs[1])
